# baseline (speedup 1.0000x reference)
import functools

import jax
import jax.numpy as jnp
from jax.experimental import pallas as pl
from jax.experimental.pallas import tpu as pltpu


def _chamfer_kernel(p1_ref, p2_ref, out_ref, row_acc, col_acc, *,
                    n1, n2, npad1, npad2, ti, tj, ni, nj):
    i = pl.program_id(0)
    j = pl.program_id(1)

    prod = jax.lax.dot_general(
        p1_ref[...], p2_ref[...],
        dimension_numbers=(((1,), (0,)), ((), ())),
        preferred_element_type=jnp.float32,
        precision=jax.lax.Precision.HIGHEST,
    )

    def mask_cols(p):
        cid = jax.lax.broadcasted_iota(jnp.int32, (ti, tj), 1) + j * tj
        return jnp.where(cid < n2, p, jnp.inf)

    def mask_rows(p):
        rid = jax.lax.broadcasted_iota(jnp.int32, (ti, tj), 0) + i * ti
        return jnp.where(rid < n1, p, jnp.inf)

    prod_r = jax.lax.cond(j == nj - 1, mask_cols, lambda p: p, prod)
    row_m = jnp.min(prod_r, axis=1)[:, None]
    prod_c = jax.lax.cond(i == ni - 1, mask_rows, lambda p: p, prod)
    col_m = jnp.min(prod_c, axis=0)[None, :]

    @pl.when(j == 0)
    def _():
        row_acc[pl.ds(i * ti, ti), :] = row_m

    @pl.when(j > 0)
    def _():
        row_acc[pl.ds(i * ti, ti), :] = jnp.minimum(
            row_acc[pl.ds(i * ti, ti), :], row_m)

    @pl.when(i == 0)
    def _():
        col_acc[:, pl.ds(j * tj, tj)] = col_m

    @pl.when(i > 0)
    def _():
        col_acc[:, pl.ds(j * tj, tj)] = jnp.minimum(
            col_acc[:, pl.ds(j * tj, tj)], col_m)

    @pl.when((i == ni - 1) & (j == nj - 1))
    def _():
        rm = row_acc[...]
        rvalid = jax.lax.broadcasted_iota(jnp.int32, (npad1, 1), 0) < n1
        s1 = jnp.sum(jnp.where(rvalid, jnp.sqrt(jnp.maximum(rm, 0.0)), 0.0))
        cm = col_acc[...]
        cvalid = jax.lax.broadcasted_iota(jnp.int32, (1, npad2), 1) < n2
        s2 = jnp.sum(jnp.where(cvalid, jnp.sqrt(jnp.maximum(cm, 0.0)), 0.0))
        out_ref[...] = (s1 + s2)[None, None]


def kernel(points1, points2):
    n1 = points1.shape[0]
    n2 = points2.shape[0]
    ti = 1024
    tj = 1024
    npad1 = ((n1 + ti - 1) // ti) * ti
    npad2 = ((n2 + tj - 1) // tj) * tj
    ni = npad1 // ti
    nj = npad2 // tj

    p1 = points1.astype(jnp.float32)
    p2 = points2.astype(jnp.float32)
    sq1 = jnp.sum(p1 * p1, axis=1, keepdims=True)
    sq2 = jnp.sum(p2 * p2, axis=1, keepdims=True)

    p1aug = jnp.zeros((npad1, 8), jnp.float32)
    p1aug = p1aug.at[:n1, :3].set(-2.0 * p1)
    p1aug = p1aug.at[:n1, 3:4].set(sq1)
    p1aug = p1aug.at[:n1, 4].set(1.0)

    p2aug = jnp.zeros((8, npad2), jnp.float32)
    p2aug = p2aug.at[:3, :n2].set(p2.T)
    p2aug = p2aug.at[3, :n2].set(1.0)
    p2aug = p2aug.at[4:5, :n2].set(sq2.T)

    body = functools.partial(
        _chamfer_kernel,
        n1=n1, n2=n2, npad1=npad1, npad2=npad2,
        ti=ti, tj=tj, ni=ni, nj=nj)

    out = pl.pallas_call(
        body,
        grid=(ni, nj),
        in_specs=[
            pl.BlockSpec((ti, 8), lambda i, j: (i, 0)),
            pl.BlockSpec((8, tj), lambda i, j: (0, j)),
        ],
        out_specs=pl.BlockSpec((1, 1), lambda i, j: (0, 0)),
        out_shape=jax.ShapeDtypeStruct((1, 1), jnp.float32),
        scratch_shapes=[
            pltpu.VMEM((npad1, 1), jnp.float32),
            pltpu.VMEM((1, npad2), jnp.float32),
        ],
        compiler_params=pltpu.CompilerParams(
            dimension_semantics=("arbitrary", "arbitrary"),
        ),
    )(p1aug, p2aug)
    return out[0, 0]


# speedup vs baseline: 2.6539x; 2.6539x over previous
import functools

import jax
import jax.numpy as jnp
from jax.experimental import pallas as pl
from jax.experimental.pallas import tpu as pltpu


def _chamfer_kernel(p1_ref, p2_ref, out_ref, row_acc, col_acc, *,
                    n1, n2, npad1, npad2, ti, tj, ni, nj):
    i = pl.program_id(0)
    j = pl.program_id(1)

    p1 = p1_ref[...]
    p2 = p2_ref[...]

    t = p2[3, :][None, :] + p1[:, 0][:, None] * p2[0, :][None, :]
    t = t + p1[:, 1][:, None] * p2[1, :][None, :]
    t = t + p1[:, 2][:, None] * p2[2, :][None, :]

    sq1 = p1[:, 3][:, None]
    row_m = jnp.min(t, axis=1)[:, None] + sq1
    col_m = jnp.min(t + sq1, axis=0)[None, :]

    @pl.when(j == 0)
    def _():
        row_acc[pl.ds(i * ti, ti), :] = row_m

    @pl.when(j > 0)
    def _():
        row_acc[pl.ds(i * ti, ti), :] = jnp.minimum(
            row_acc[pl.ds(i * ti, ti), :], row_m)

    @pl.when(i == 0)
    def _():
        col_acc[:, pl.ds(j * tj, tj)] = col_m

    @pl.when(i > 0)
    def _():
        col_acc[:, pl.ds(j * tj, tj)] = jnp.minimum(
            col_acc[:, pl.ds(j * tj, tj)], col_m)

    @pl.when((i == ni - 1) & (j == nj - 1))
    def _():
        rm = row_acc[...]
        rvalid = jax.lax.broadcasted_iota(jnp.int32, (npad1, 1), 0) < n1
        s1 = jnp.sum(jnp.where(rvalid, jnp.sqrt(jnp.maximum(rm, 0.0)), 0.0))
        cm = col_acc[...]
        cvalid = jax.lax.broadcasted_iota(jnp.int32, (1, npad2), 1) < n2
        s2 = jnp.sum(jnp.where(cvalid, jnp.sqrt(jnp.maximum(cm, 0.0)), 0.0))
        out_ref[...] = (s1 + s2)[None, None]


def kernel(points1, points2):
    n1 = points1.shape[0]
    n2 = points2.shape[0]
    ti = 1024
    tj = 1024
    npad1 = ((n1 + ti - 1) // ti) * ti
    npad2 = ((n2 + tj - 1) // tj) * tj
    ni = npad1 // ti
    nj = npad2 // tj

    p1 = points1.astype(jnp.float32)
    p2 = points2.astype(jnp.float32)
    sq1 = jnp.sum(p1 * p1, axis=1, keepdims=True)
    sq2 = jnp.sum(p2 * p2, axis=1, keepdims=True)

    p1a = jnp.zeros((npad1, 8), jnp.float32)
    p1a = p1a.at[:, 3].set(jnp.inf)
    p1a = p1a.at[:n1, :3].set(-2.0 * p1)
    p1a = p1a.at[:n1, 3:4].set(sq1)

    p2a = jnp.zeros((8, npad2), jnp.float32)
    p2a = p2a.at[3, :].set(jnp.inf)
    p2a = p2a.at[:3, :n2].set(p2.T)
    p2a = p2a.at[3:4, :n2].set(sq2.T)

    body = functools.partial(
        _chamfer_kernel,
        n1=n1, n2=n2, npad1=npad1, npad2=npad2,
        ti=ti, tj=tj, ni=ni, nj=nj)

    out = pl.pallas_call(
        body,
        grid=(ni, nj),
        in_specs=[
            pl.BlockSpec((ti, 8), lambda i, j: (i, 0)),
            pl.BlockSpec((8, tj), lambda i, j: (0, j)),
        ],
        out_specs=pl.BlockSpec((1, 1), lambda i, j: (0, 0)),
        out_shape=jax.ShapeDtypeStruct((1, 1), jnp.float32),
        scratch_shapes=[
            pltpu.VMEM((npad1, 1), jnp.float32),
            pltpu.VMEM((1, npad2), jnp.float32),
        ],
        compiler_params=pltpu.CompilerParams(
            dimension_semantics=("arbitrary", "arbitrary"),
        ),
    )(p1a, p2a)
    return out[0, 0]
